# Initial kernel scaffold; baseline (speedup 1.0000x reference)
import jax
import jax.numpy as jnp
from jax.experimental import pallas as pl


def kernel(x, table):
    raise NotImplementedError("write your pallas kernel here")


# speedup vs baseline: 1.6745x; 1.6745x over previous
import functools

import jax
import jax.numpy as jnp
import numpy as np
from jax import lax
from jax.experimental import pallas as pl
from jax.experimental.pallas import tpu as pltpu
from jax.experimental.pallas import tpu_sc as plsc

VOCAB = 100000
MAX_LEN = 200
EMBED = 64
B = 1024
L = 200

NUM_CORES = 2
NUM_SUBCORES = 16
NUM_WORKERS = NUM_CORES * NUM_SUBCORES
ROWS = B * L
ROWS_PER_W = ROWS // NUM_WORKERS
CHUNK = 40
CHUNKS_PER_W = ROWS_PER_W // CHUNK
LANES = 16


def _positional_encoding(length, depth):
    depth = depth / 2
    positions = np.arange(length)[:, np.newaxis]
    depths = np.arange(depth)[np.newaxis, :] / depth
    angle_rates = 1 / 10000.0 ** depths
    angle_rads = positions * angle_rates
    enc = np.concatenate([np.sin(angle_rads), np.cos(angle_rads)], axis=-1)
    return enc.astype(np.float32)


_POS = _positional_encoding(MAX_LEN, EMBED)
_SCALE = float(np.sqrt(EMBED))


def _sc_body(x_hbm, pos_hbm, table_hbm, out_hbm, idx_v, pos_v, rows_v, sem):
    c = lax.axis_index("c")
    s = lax.axis_index("s")
    wid = s * NUM_CORES + c
    base = wid * ROWS_PER_W

    pltpu.sync_copy(x_hbm.at[pl.ds(base, ROWS_PER_W)], idx_v)
    pltpu.sync_copy(pos_hbm, pos_v)

    @pl.loop(0, CHUNKS_PER_W)
    def _chunk(ci):
        r0 = ci * CHUNK
        l0 = lax.rem(r0, L)
        pltpu.async_copy(
            table_hbm.at[idx_v.at[pl.ds(r0, CHUNK)]], rows_v, sem
        ).wait()

        @pl.loop(0, CHUNK)
        def _row(i):
            for j in range(EMBED // LANES):
                sl = pl.ds(j * LANES, LANES)
                rows_v[i, sl] = rows_v[i, sl] * _SCALE + pos_v[l0 + i, sl]

        pltpu.sync_copy(rows_v, out_hbm.at[pl.ds(base + r0, CHUNK)])


@functools.partial(
    pl.kernel,
    out_type=jax.ShapeDtypeStruct((ROWS, EMBED), jnp.float32),
    mesh=plsc.VectorSubcoreMesh(core_axis_name="c", subcore_axis_name="s"),
    compiler_params=pltpu.CompilerParams(use_tc_tiling_on_sc=False),
    scratch_types=[
        pltpu.VMEM((ROWS_PER_W,), jnp.int32),
        pltpu.VMEM((MAX_LEN, EMBED), jnp.float32),
        pltpu.VMEM((CHUNK, EMBED), jnp.float32),
        pltpu.SemaphoreType.DMA,
    ],
)
def _sc_embed(x_hbm, pos_hbm, table_hbm, out_hbm, idx_v, pos_v, rows_v, sem):
    _sc_body(x_hbm, pos_hbm, table_hbm, out_hbm, idx_v, pos_v, rows_v, sem)


def kernel(x, table):
    pos = jnp.asarray(_POS)
    flat = _sc_embed(x.reshape(ROWS), pos, table)
    return flat.reshape(B, L, EMBED)


# speedup vs baseline: 3.2072x; 1.9153x over previous
import functools

import jax
import jax.numpy as jnp
import numpy as np
from jax import lax
from jax.experimental import pallas as pl
from jax.experimental.pallas import tpu as pltpu
from jax.experimental.pallas import tpu_sc as plsc

VOCAB = 100000
MAX_LEN = 200
EMBED = 64
B = 1024
L = 200

NUM_CORES = 2
NUM_SUBCORES = 16
NUM_WORKERS = NUM_CORES * NUM_SUBCORES
ROWS = B * L
ROWS_PER_W = ROWS // NUM_WORKERS
BATCHES_PER_W = ROWS_PER_W // L
GCHUNK = 40
GPER = L // GCHUNK
LANES = 16
NBUF = 2


def _positional_encoding(length, depth):
    depth = depth / 2
    positions = np.arange(length)[:, np.newaxis]
    depths = np.arange(depth)[np.newaxis, :] / depth
    angle_rates = 1 / 10000.0 ** depths
    angle_rads = positions * angle_rates
    enc = np.concatenate([np.sin(angle_rads), np.cos(angle_rads)], axis=-1)
    return enc.astype(np.float32)


_POS = _positional_encoding(MAX_LEN, EMBED)
_SCALE = float(np.sqrt(EMBED))


def _sc_body(x_hbm, pos_hbm, table_hbm, out_hbm,
             idx_v, pos_v, gbufs, obufs, gsems, osems):
    c = lax.axis_index("c")
    s = lax.axis_index("s")
    wid = s * NUM_CORES + c
    base = wid * ROWS_PER_W

    pltpu.sync_copy(x_hbm.at[pl.ds(base, ROWS_PER_W)], idx_v)
    pltpu.sync_copy(pos_hbm, pos_v)

    def fire_gathers(k, b):
        for j in range(GPER):
            pltpu.async_copy(
                table_hbm.at[idx_v.at[pl.ds(k * L + j * GCHUNK, GCHUNK)]],
                gbufs[b].at[pl.ds(j * GCHUNK, GCHUNK)],
                gsems[b],
            )

    def wait_gathers(k, b):
        for j in range(GPER):
            pltpu.make_async_copy(
                table_hbm.at[idx_v.at[pl.ds(k * L + j * GCHUNK, GCHUNK)]],
                gbufs[b].at[pl.ds(j * GCHUNK, GCHUNK)],
                gsems[b],
            ).wait()

    def out_slice(k):
        return out_hbm.at[pl.ds(base + k * L, L)]

    for b in range(NBUF):
        fire_gathers(b, b)

    @pl.loop(0, BATCHES_PER_W, step=NBUF)
    def _group(k0):
        for b in range(NBUF):
            k = k0 + b
            wait_gathers(k, b)

            @pl.when(k >= NBUF)
            def _():
                pltpu.make_async_copy(obufs[b], out_slice(k - NBUF),
                                      osems[b]).wait()

            @pl.loop(0, L)
            def _row(i):
                for j in range(EMBED // LANES):
                    sl = pl.ds(j * LANES, LANES)
                    obufs[b][i, sl] = gbufs[b][i, sl] * _SCALE + pos_v[i, sl]

            @pl.when(k + NBUF < BATCHES_PER_W)
            def _():
                fire_gathers(k + NBUF, b)

            pltpu.async_copy(obufs[b], out_slice(k), osems[b])

    for b in range(NBUF):
        k = BATCHES_PER_W - NBUF + b
        pltpu.make_async_copy(obufs[b], out_slice(k), osems[b]).wait()


@functools.partial(
    pl.kernel,
    out_type=jax.ShapeDtypeStruct((ROWS, EMBED), jnp.float32),
    mesh=plsc.VectorSubcoreMesh(core_axis_name="c", subcore_axis_name="s"),
    compiler_params=pltpu.CompilerParams(use_tc_tiling_on_sc=False),
    scratch_types=[
        pltpu.VMEM((ROWS_PER_W,), jnp.int32),
        pltpu.VMEM((MAX_LEN, EMBED), jnp.float32),
        [pltpu.VMEM((L, EMBED), jnp.float32) for _ in range(NBUF)],
        [pltpu.VMEM((L, EMBED), jnp.float32) for _ in range(NBUF)],
        [pltpu.SemaphoreType.DMA for _ in range(NBUF)],
        [pltpu.SemaphoreType.DMA for _ in range(NBUF)],
    ],
)
def _sc_embed(x_hbm, pos_hbm, table_hbm, out_hbm,
              idx_v, pos_v, gbufs, obufs, gsems, osems):
    _sc_body(x_hbm, pos_hbm, table_hbm, out_hbm,
             idx_v, pos_v, gbufs, obufs, gsems, osems)


def kernel(x, table):
    pos = jnp.asarray(_POS)
    flat = _sc_embed(x.reshape(ROWS), pos, table)
    return flat.reshape(B, L, EMBED)
